# Initial kernel scaffold; baseline (speedup 1.0000x reference)
import jax
import jax.numpy as jnp
from jax.experimental import pallas as pl


def kernel(y_pr, mask):
    raise NotImplementedError("write your pallas kernel here")


# speedup vs baseline: 22.8934x; 22.8934x over previous
import functools

import jax
import jax.numpy as jnp
from jax import lax
from jax.experimental import pallas as pl
from jax.experimental.pallas import tpu as pltpu
from jax.experimental.pallas import tpu_sc as plsc

_B = 4
_H = 512
_W = 512
_NPB = _H * _W
_TOT = _B * _NPB
_NC = 2
_NS = 16
_NW = _NC * _NS
_CHUNK = _TOT // _NW
_CPB = _NPB // _CHUNK
_ROWS = _CHUNK // _W
_NVEC = _CHUNK // 16
_NLAB = 30

_CD = {2: 18, 3: 18, 4: 18.5, 5: 19, 6: 19.5, 7: 20, 8: 20, 9: 20,
       10: 20.5, 11: 21, 12: 21.5, 13: 22, 14: 22.5, 15: 23, 16: 24.5,
       17: 24.5, 18: 26.5, 19: 28.5, 20: 29.5, 21: 33, 22: 33, 23: 33,
       24: 33, 25: 33, 26: 33}
_MD = tuple(float(_CD[i]) if i in _CD else (30.0 if i > 26 else 14.0)
            for i in range(_NLAB))


def _sc_histogram(yf, mf):
    mesh = plsc.VectorSubcoreMesh(core_axis_name="c", subcore_axis_name="s",
                                  num_cores=_NC, num_subcores=_NS)

    @functools.partial(
        pl.kernel,
        out_type=[
            jax.ShapeDtypeStruct((_NW * 3 * _NLAB * 16,), jnp.int32),
            jax.ShapeDtypeStruct((_NW * 32,), jnp.float32),
        ],
        mesh=mesh,
        scratch_types=[
            pltpu.VMEM((_CHUNK,), jnp.float32),
            pltpu.VMEM((_CHUNK,), jnp.float32),
            pltpu.VMEM((_NLAB * 16,), jnp.int32),
            pltpu.VMEM((_NLAB * 16,), jnp.int32),
            pltpu.VMEM((_NLAB * 16,), jnp.int32),
            pltpu.VMEM((32,), jnp.float32),
        ],
        compiler_params=pltpu.CompilerParams(needs_layout_passes=False),
    )
    def hist_kernel(y_hbm, m_hbm, out_i, out_f, yv, mv, cnt, ysum, xsum, mmv):
        cid = lax.axis_index("c")
        sid = lax.axis_index("s")
        wid = cid * _NS + sid
        base = wid * _CHUNK
        pltpu.sync_copy(y_hbm.at[pl.ds(base, _CHUNK)], yv)
        pltpu.sync_copy(m_hbm.at[pl.ds(base, _CHUNK)], mv)

        zero = jnp.zeros((16,), jnp.int32)

        def zinit(i, c):
            off = i * 16
            cnt[pl.ds(off, 16)] = zero
            ysum[pl.ds(off, 16)] = zero
            xsum[pl.ds(off, 16)] = zero
            return c

        lax.fori_loop(0, _NLAB, zinit, 0)

        lanes = lax.iota(jnp.int32, 16)
        ones = jnp.ones((16,), jnp.int32)
        row0 = (wid % _CPB) * _ROWS

        def step(j, carry):
            mn, mx = carry
            off = j * 16
            y = yv[pl.ds(off, 16)]
            m = mv[pl.ds(off, 16)]
            v = y * m
            lbl = v.astype(jnp.int32)
            addr = lbl * 16 + lanes
            row = row0 + lax.shift_right_logical(off, 9)
            col0 = lax.bitwise_and(off, _W - 1)
            colv = lanes + col0
            rowv = jnp.full((16,), row, jnp.int32)
            plsc.addupdate_scatter(cnt, [addr], ones)
            plsc.addupdate_scatter(ysum, [addr], rowv)
            plsc.addupdate_scatter(xsum, [addr], colv)
            return jnp.minimum(mn, v), jnp.maximum(mx, v)

        inf = jnp.full((16,), jnp.inf, jnp.float32)
        mn, mx = lax.fori_loop(0, _NVEC, step, (inf, -inf))
        mmv[pl.ds(0, 16)] = mn
        mmv[pl.ds(16, 16)] = mx
        hbase = wid * 3 * _NLAB * 16
        pltpu.sync_copy(cnt, out_i.at[pl.ds(hbase, _NLAB * 16)])
        pltpu.sync_copy(ysum, out_i.at[pl.ds(hbase + _NLAB * 16, _NLAB * 16)])
        pltpu.sync_copy(xsum, out_i.at[pl.ds(hbase + 2 * _NLAB * 16, _NLAB * 16)])
        pltpu.sync_copy(mmv, out_f.at[pl.ds(wid * 32, 32)])

    return hist_kernel(yf, mf)


def _shift_right(a, s, fill):
    pad = jnp.full(a.shape[:-1] + (s,), fill, a.dtype)
    return jnp.concatenate([pad, a[..., :-s]], axis=-1)


def _tc_chain(hist, mm):
    def chain_kernel(h_ref, mm_ref, md_ref, o_ref):
        sums = jnp.sum(h_ref[...], axis=-1)
        cntf = sums[:, 0, :].astype(jnp.float32)
        ysf = sums[:, 1, :].astype(jnp.float32)
        xsf = sums[:, 2, :].astype(jnp.float32)
        mmval = mm_ref[...]
        mn = jnp.min(mmval[:, 0, :], axis=-1, keepdims=True)
        mx = jnp.max(mmval[:, 1, :], axis=-1, keepdims=True)

        ilab = lax.broadcasted_iota(jnp.int32, (_B, _NLAB), 1)
        fi = ilab.astype(jnp.float32)
        nonempty = cntf > 0.5
        active = (fi >= mn + 1.0) & (fi <= mx)
        valid = active & (ilab >= 1)
        visit = active & nonempty
        denom = jnp.where(nonempty, cntf, 1.0)
        yc = ysf / denom
        xc = xsf / denom

        fy = _shift_right(yc, 1, 0.0)
        fx = _shift_right(xc, 1, 0.0)
        fne = _shift_right(nonempty.astype(jnp.float32), 1, 0.0)
        fv = _shift_right(valid.astype(jnp.float32), 1, 0.0)
        fne = jnp.where(fv > 0.5, fne, 0.0)
        fy = jnp.where(fv > 0.5, fy, 0.0)
        fx = jnp.where(fv > 0.5, fx, 0.0)
        s = 1
        while s < _NLAB:
            keep = fv > 0.5
            fy = jnp.where(keep, fy, _shift_right(fy, s, 0.0))
            fx = jnp.where(keep, fx, _shift_right(fx, s, 0.0))
            fne = jnp.where(keep, fne, _shift_right(fne, s, 0.0))
            fv = jnp.maximum(fv, _shift_right(fv, s, 0.0))
            s *= 2

        has_prev = (fv > 0.5) & (fne > 0.5)
        md = md_ref[...]
        dist = jnp.sqrt((xc - fx) ** 2 + (yc - fy) ** 2)
        term = jnp.abs(dist - md)
        contrib = jnp.where(visit & has_prev & (ilab >= 1), term, 0.0)
        o_ref[...] = jnp.sum(contrib)[None, None]

    md = jnp.asarray(_MD, jnp.float32)[None, :]
    return pl.pallas_call(
        chain_kernel,
        out_shape=jax.ShapeDtypeStruct((1, 1), jnp.float32),
    )(hist, mm, md)


def kernel(y_pr, mask):
    yf = y_pr.reshape(_TOT)
    mf = mask.reshape(_TOT)
    out_i, out_f = _sc_histogram(yf, mf)
    hist = (out_i.reshape(_B, _CPB, 3, _NLAB, 16)
            .transpose(0, 2, 3, 1, 4)
            .reshape(_B, 3, _NLAB, _CPB * 16))
    mm = (out_f.reshape(_B, _CPB, 2, 16)
          .transpose(0, 2, 1, 3)
          .reshape(_B, 2, _CPB * 16))
    loss = _tc_chain(hist, mm)
    return loss[0, 0]


# speedup vs baseline: 24.5005x; 1.0702x over previous
import functools

import jax
import jax.numpy as jnp
from jax import lax
from jax.experimental import pallas as pl
from jax.experimental.pallas import tpu as pltpu
from jax.experimental.pallas import tpu_sc as plsc

_B = 4
_H = 512
_W = 512
_NPB = _H * _W
_TOT = _B * _NPB
_NC = 2
_NS = 16
_NW = _NC * _NS
_CHUNK = _TOT // _NW
_CPB = _NPB // _CHUNK
_ROWS = _CHUNK // _W
_NVEC = _CHUNK // 16
_NLAB = 30
_HSZ = _NLAB * 16
_PACK = 1 << 20

_CD = {2: 18, 3: 18, 4: 18.5, 5: 19, 6: 19.5, 7: 20, 8: 20, 9: 20,
       10: 20.5, 11: 21, 12: 21.5, 13: 22, 14: 22.5, 15: 23, 16: 24.5,
       17: 24.5, 18: 26.5, 19: 28.5, 20: 29.5, 21: 33, 22: 33, 23: 33,
       24: 33, 25: 33, 26: 33}
_MD = tuple(float(_CD[i]) if i in _CD else (30.0 if i > 26 else 14.0)
            for i in range(_NLAB))


def _sc_histogram(yf, mf):
    mesh = plsc.VectorSubcoreMesh(core_axis_name="c", subcore_axis_name="s",
                                  num_cores=_NC, num_subcores=_NS)

    @functools.partial(
        pl.kernel,
        out_type=jax.ShapeDtypeStruct((_NW * 2 * _HSZ,), jnp.int32),
        mesh=mesh,
        scratch_types=[
            pltpu.VMEM((_CHUNK,), jnp.float32),
            pltpu.VMEM((_CHUNK,), jnp.float32),
            pltpu.VMEM((_HSZ,), jnp.int32),
            pltpu.VMEM((_HSZ,), jnp.int32),
        ],
        compiler_params=pltpu.CompilerParams(needs_layout_passes=False),
    )
    def hist_kernel(y_hbm, m_hbm, out_i, yv, mv, cx, ys):
        cid = lax.axis_index("c")
        sid = lax.axis_index("s")
        wid = cid * _NS + sid
        base = wid * _CHUNK
        pltpu.sync_copy(y_hbm.at[pl.ds(base, _CHUNK)], yv)
        pltpu.sync_copy(m_hbm.at[pl.ds(base, _CHUNK)], mv)

        zero = jnp.zeros((16,), jnp.int32)

        def zinit(i, c):
            off = i * 16
            cx[pl.ds(off, 16)] = zero
            ys[pl.ds(off, 16)] = zero
            return c

        lax.fori_loop(0, _NLAB, zinit, 0, unroll=6)

        lanes = lax.iota(jnp.int32, 16)
        row0 = (wid % _CPB) * _ROWS

        def step(j, c):
            off = j * 16
            y = yv[pl.ds(off, 16)]
            m = mv[pl.ds(off, 16)]
            v = y * m
            lbl = v.astype(jnp.int32)
            addr = lbl * 16 + lanes
            row = row0 + lax.shift_right_logical(off, 9)
            col0 = lax.bitwise_and(off, _W - 1)
            packed = lanes + (col0 + _PACK)
            rowv = jnp.full((16,), row, jnp.int32)
            plsc.addupdate_scatter(cx, [addr], packed)
            plsc.addupdate_scatter(ys, [addr], rowv)
            return c

        lax.fori_loop(0, _NVEC, step, 0, unroll=8)

        hbase = wid * 2 * _HSZ
        pltpu.sync_copy(cx, out_i.at[pl.ds(hbase, _HSZ)])
        pltpu.sync_copy(ys, out_i.at[pl.ds(hbase + _HSZ, _HSZ)])

    return hist_kernel(yf, mf)


def _shift_right(a, s, fill):
    pad = jnp.full(a.shape[:-1] + (s,), fill, a.dtype)
    return jnp.concatenate([pad, a[..., :-s]], axis=-1)


def _tc_chain(hist):
    def chain_kernel(h_ref, md_ref, o_ref):
        h = h_ref[...]
        packed = h[:, 0, :, :]
        cnt128 = lax.shift_right_logical(packed, 20)
        xs128 = lax.bitwise_and(packed, _PACK - 1)
        cnt = jnp.sum(cnt128, axis=-1)
        xsum = jnp.sum(xs128, axis=-1)
        ysum = jnp.sum(h[:, 1, :, :], axis=-1)
        cntf = cnt.astype(jnp.float32)
        ysf = ysum.astype(jnp.float32)
        xsf = xsum.astype(jnp.float32)

        ilab = lax.broadcasted_iota(jnp.int32, (_B, _NLAB), 1)
        fi = ilab.astype(jnp.float32)
        nonempty = cnt > 0
        mn = jnp.min(jnp.where(nonempty, fi, 1e9), axis=-1, keepdims=True)
        mx = jnp.max(jnp.where(nonempty, fi, -1e9), axis=-1, keepdims=True)

        active = (fi >= mn + 1.0) & (fi <= mx)
        valid = active & (ilab >= 1)
        visit = active & nonempty
        denom = jnp.where(nonempty, cntf, 1.0)
        yc = ysf / denom
        xc = xsf / denom

        fy = _shift_right(yc, 1, 0.0)
        fx = _shift_right(xc, 1, 0.0)
        fne = _shift_right(nonempty.astype(jnp.float32), 1, 0.0)
        fv = _shift_right(valid.astype(jnp.float32), 1, 0.0)
        fne = jnp.where(fv > 0.5, fne, 0.0)
        fy = jnp.where(fv > 0.5, fy, 0.0)
        fx = jnp.where(fv > 0.5, fx, 0.0)
        s = 1
        while s < _NLAB:
            keep = fv > 0.5
            fy = jnp.where(keep, fy, _shift_right(fy, s, 0.0))
            fx = jnp.where(keep, fx, _shift_right(fx, s, 0.0))
            fne = jnp.where(keep, fne, _shift_right(fne, s, 0.0))
            fv = jnp.maximum(fv, _shift_right(fv, s, 0.0))
            s *= 2

        has_prev = (fv > 0.5) & (fne > 0.5)
        md = md_ref[...]
        dist = jnp.sqrt((xc - fx) ** 2 + (yc - fy) ** 2)
        term = jnp.abs(dist - md)
        contrib = jnp.where(visit & has_prev & (ilab >= 1), term, 0.0)
        o_ref[...] = jnp.sum(contrib)[None, None]

    md = jnp.asarray(_MD, jnp.float32)[None, :]
    return pl.pallas_call(
        chain_kernel,
        out_shape=jax.ShapeDtypeStruct((1, 1), jnp.float32),
    )(hist, md)


def kernel(y_pr, mask):
    yf = y_pr.reshape(_TOT)
    mf = mask.reshape(_TOT)
    out_i = _sc_histogram(yf, mf)
    hist = (out_i.reshape(_B, _CPB, 2, _NLAB, 16)
            .transpose(0, 2, 3, 1, 4)
            .reshape(_B, 2, _NLAB, _CPB * 16))
    loss = _tc_chain(hist)
    return loss[0, 0]


# speedup vs baseline: 44.6399x; 1.8220x over previous
import functools

import numpy as np

import jax
import jax.numpy as jnp
from jax import lax
from jax.experimental import pallas as pl
from jax.experimental.pallas import tpu as pltpu
from jax.experimental.pallas import tpu_sc as plsc

_B = 4
_H = 512
_W = 512
_NPB = _H * _W
_TOT = _B * _NPB
_NC = 2
_NS = 16
_NW = _NC * _NS
_CHUNK = _TOT // _NW
_CPB = _NPB // _CHUNK
_ROWS = _CHUNK // _W
_NVEC = _CHUNK // 16
_NLAB = 30
_HSZ = _NLAB * 16
_PACK = 1 << 20

_CD = {2: 18, 3: 18, 4: 18.5, 5: 19, 6: 19.5, 7: 20, 8: 20, 9: 20,
       10: 20.5, 11: 21, 12: 21.5, 13: 22, 14: 22.5, 15: 23, 16: 24.5,
       17: 24.5, 18: 26.5, 19: 28.5, 20: 29.5, 21: 33, 22: 33, 23: 33,
       24: 33, 25: 33, 26: 33}
_MD = tuple(float(_CD[i]) if i in _CD else (30.0 if i > 26 else 14.0)
            for i in range(_NLAB))
_MD_NP = np.asarray(_MD, np.float32)[None, :]
_BM_NP = (np.arange(4)[:, None] == (np.arange(32)[None, :] // 8)
          ).astype(np.float32)
_LM_NP = (np.arange(32)[None, :] == (np.arange(30 * 16)[:, None] // 16)
          ).astype(np.float32)


_GRP = 4
_GROWS = _ROWS // _GRP
_GVEC = _GROWS * _W // 16


def _sc_histogram(y2, m2):
    mesh = plsc.VectorSubcoreMesh(core_axis_name="c", subcore_axis_name="s",
                                  num_cores=_NC, num_subcores=_NS)

    @functools.partial(
        pl.kernel,
        out_type=jax.ShapeDtypeStruct((_NW * 2 * _HSZ,), jnp.int32),
        mesh=mesh,
        scratch_types=[
            pltpu.VMEM((_GROWS, _W), jnp.float32),
            pltpu.VMEM((_GROWS, _W), jnp.float32),
            pltpu.VMEM((_GROWS, _W), jnp.float32),
            pltpu.VMEM((_GROWS, _W), jnp.float32),
            pltpu.VMEM((_HSZ,), jnp.int32),
            pltpu.VMEM((_HSZ,), jnp.int32),
            pltpu.SemaphoreType.DMA,
            pltpu.SemaphoreType.DMA,
        ],
        compiler_params=pltpu.CompilerParams(needs_layout_passes=False),
    )
    def hist_kernel(y_hbm, m_hbm, out_i, ya, yb, ma, mb, cx, ys, sa, sb):
        cid = lax.axis_index("c")
        sid = lax.axis_index("s")
        wid = cid * _NS + sid
        grow0 = wid * _ROWS
        ybufs = (ya, yb)
        mbufs = (ma, mb)
        sems = (sa, sb)

        def start(g):
            r0 = grow0 + g * _GROWS
            s = sems[g % 2]
            dy = pltpu.make_async_copy(
                y_hbm.at[pl.ds(r0, _GROWS), :], ybufs[g % 2], s)
            dm = pltpu.make_async_copy(
                m_hbm.at[pl.ds(r0, _GROWS), :], mbufs[g % 2], s)
            dy.start()
            dm.start()
            return dy, dm

        descs = start(0)

        zero = jnp.zeros((16,), jnp.int32)

        @plsc.parallel_loop(0, _NLAB, unroll=6)
        def zinit(i):
            off = i * 16
            cx[pl.ds(off, 16)] = zero
            ys[pl.ds(off, 16)] = zero

        lanes = lax.iota(jnp.int32, 16)

        for g in range(_GRP):
            nxt = start(g + 1) if g + 1 < _GRP else None
            descs[0].wait()
            descs[1].wait()
            yv = ybufs[g % 2]
            mv = mbufs[g % 2]
            rbase = (grow0 + g * _GROWS) % _H

            @plsc.parallel_loop(0, _GVEC, unroll=8)
            def step(j):
                r = lax.shift_right_logical(j, 5)
                c = lax.bitwise_and(j, 31) * 16
                y = yv[r, pl.ds(c, 16)]
                m = mv[r, pl.ds(c, 16)]
                v = y * m
                lbl = v.astype(jnp.int32)
                addr = lbl * 16 + lanes
                packed = lanes + (c + _PACK)
                rowv = jnp.full((16,), rbase + r, jnp.int32)
                plsc.addupdate_scatter(cx, [addr], packed)
                plsc.addupdate_scatter(ys, [addr], rowv)

            descs = nxt

        pltpu.sync_copy(cx, out_i.at[pl.ds(wid * _HSZ, _HSZ)])
        pltpu.sync_copy(ys, out_i.at[pl.ds((_NW + wid) * _HSZ, _HSZ)])

    return hist_kernel(y2, m2)


def _shift_right(a, s, fill):
    pad = jnp.full(a.shape[:-1] + (s,), fill, a.dtype)
    return jnp.concatenate([pad, a[..., :-s]], axis=-1)


def _tc_chain(hraw):
    def chain_kernel(h_ref, bm_ref, lm_ref, md_ref, o_ref):
        x = h_ref[...]
        p = x[:_NW, :]
        cntw = lax.shift_right_logical(p, 20).astype(jnp.float32)
        xsw = lax.bitwise_and(p, _PACK - 1).astype(jnp.float32)
        ysw = x[_NW:, :].astype(jnp.float32)
        bm = bm_ref[...]
        lm = lm_ref[...]
        hi = lax.Precision.HIGHEST

        def reduce2(a):
            return jnp.dot(jnp.dot(bm, a, precision=hi), lm,
                           precision=hi)[:, :_NLAB]

        cntf = reduce2(cntw)
        xsf = reduce2(xsw)
        ysf = reduce2(ysw)

        ilab = lax.broadcasted_iota(jnp.int32, (_B, _NLAB), 1)
        fi = ilab.astype(jnp.float32)
        nonempty = cntf > 0.5
        mn = jnp.min(jnp.where(nonempty, fi, 1e9), axis=-1, keepdims=True)
        mx = jnp.max(jnp.where(nonempty, fi, -1e9), axis=-1, keepdims=True)

        active = (fi >= mn + 1.0) & (fi <= mx)
        valid = active & (ilab >= 1)
        visit = active & nonempty
        denom = jnp.maximum(cntf, 1.0)
        yc = ysf / denom
        xc = xsf / denom

        fy = _shift_right(yc, 1, 0.0)
        fx = _shift_right(xc, 1, 0.0)
        fne = _shift_right(nonempty.astype(jnp.float32), 1, 0.0)
        fv = _shift_right(valid.astype(jnp.float32), 1, 0.0)
        fne = jnp.where(fv > 0.5, fne, 0.0)
        fy = jnp.where(fv > 0.5, fy, 0.0)
        fx = jnp.where(fv > 0.5, fx, 0.0)
        s = 1
        while s < _NLAB:
            keep = fv > 0.5
            fy = jnp.where(keep, fy, _shift_right(fy, s, 0.0))
            fx = jnp.where(keep, fx, _shift_right(fx, s, 0.0))
            fne = jnp.where(keep, fne, _shift_right(fne, s, 0.0))
            fv = jnp.maximum(fv, _shift_right(fv, s, 0.0))
            s *= 2

        has_prev = (fv > 0.5) & (fne > 0.5)
        md = md_ref[...]
        dist = jnp.sqrt((xc - fx) ** 2 + (yc - fy) ** 2)
        term = jnp.abs(dist - md)
        contrib = jnp.where(visit & has_prev & (ilab >= 1), term, 0.0)
        o_ref[...] = jnp.sum(contrib)[None, None]

    md = jnp.asarray(_MD_NP)
    bm = jnp.asarray(_BM_NP)
    lm = jnp.asarray(_LM_NP)
    return pl.pallas_call(
        chain_kernel,
        out_shape=jax.ShapeDtypeStruct((1, 1), jnp.float32),
    )(hraw, bm, lm, md)


def kernel(y_pr, mask):
    yf = y_pr.reshape(_B * _H, _W)
    mf = mask.reshape(_B * _H, _W)
    out_i = _sc_histogram(yf, mf)
    loss = _tc_chain(out_i.reshape(2 * _NW, _HSZ))
    return loss[0, 0]
